# baseline (device time: 148635 ns/iter reference)
import jax
import jax.numpy as jnp
from jax import lax
from jax.experimental import pallas as pl
from jax.experimental.pallas import tpu as pltpu

B, S, D = 1, 1024, 2048
H, Dh, Dr = 16, 128, 32
DC_HALF = 128
DC = 256
NROW = 4
SR = S // NROW
NCK = 2
CH = SR // NCK


def kernel(x, Wdkv, Wuk, Wuv, Wq, Wqr, Wkr, Wo):
    WukT = Wuk.reshape(DC_HALF, H, Dh).transpose(1, 0, 2)
    WuvT = Wuv.reshape(DC_HALF, H, Dh).transpose(1, 0, 2)
    WqrT = Wqr.reshape(D, H, Dr).transpose(1, 0, 2)

    def body(
        x_ref, wdkv_ref, wuk_ref, wuv_ref, wq_ref, wqr_ref, wkr_ref, wo_ref,
        out_ref,
        x_vmem, acc, c_buf, c_cat, wuk_cat, wuv_cat, k_buf, kr_buf,
        x_sem, local_sems, c_send_sem, c_recv_sem,
        wk_send_sems, wk_recv_sems, wv_send_sems, wv_recv_sems,
        ag_send_sems, ag_recv_sems,
    ):
        ck = pl.program_id(0)
        h = pl.program_id(1)
        my_x = lax.axis_index("x")
        my_y = lax.axis_index("y")
        rid = 2 * my_x + my_y
        x_nbr = (1 - my_x, my_y)
        y_nbr = (my_x, 1 - my_y)
        diag = (1 - my_x, 1 - my_y)
        peers = (x_nbr, y_nbr, diag)

        def wuk_rdma(hh):
            return pltpu.make_async_remote_copy(
                src_ref=wuk_ref.at[hh],
                dst_ref=wuk_cat.at[hh, DC_HALF:, :],
                send_sem=wk_send_sems.at[hh], recv_sem=wk_recv_sems.at[hh],
                device_id=x_nbr, device_id_type=pl.DeviceIdType.MESH,
            )

        def wuv_rdma(hh):
            return pltpu.make_async_remote_copy(
                src_ref=wuv_ref.at[hh],
                dst_ref=wuv_cat.at[hh, DC_HALF:, :],
                send_sem=wv_send_sems.at[hh], recv_sem=wv_recv_sems.at[hh],
                device_id=x_nbr, device_id_type=pl.DeviceIdType.MESH,
            )

        def c_rdma():
            return pltpu.make_async_remote_copy(
                src_ref=c_buf, dst_ref=c_cat.at[:, DC_HALF:],
                send_sem=c_send_sem, recv_sem=c_recv_sem,
                device_id=x_nbr, device_id_type=pl.DeviceIdType.MESH,
            )

        def ag_rdma(cck, i, p):
            return pltpu.make_async_remote_copy(
                src_ref=acc.at[pl.ds(cck * CH, CH), :],
                dst_ref=out_ref.at[rid, pl.ds(cck * CH, CH), :],
                send_sem=ag_send_sems.at[cck, i],
                recv_sem=ag_recv_sems.at[cck, i],
                device_id=p, device_id_type=pl.DeviceIdType.MESH,
            )

        def local_copy(cck):
            return pltpu.make_async_copy(
                acc.at[pl.ds(cck * CH, CH), :],
                out_ref.at[rid, pl.ds(cck * CH, CH), :],
                local_sems.at[cck],
            )

        @pl.when(jnp.logical_and(ck == 0, h == 0))
        def _exchange():
            barrier_sem = pltpu.get_barrier_semaphore()
            for p in peers:
                pl.semaphore_signal(
                    barrier_sem, inc=1, device_id=p,
                    device_id_type=pl.DeviceIdType.MESH,
                )
            pl.semaphore_wait(barrier_sem, 3)

            for hh in range(H):
                wuk_rdma(hh).start()
                wuv_rdma(hh).start()

            xcopy = pltpu.make_async_copy(x_ref.at[0], x_vmem, x_sem)
            xcopy.start()
            xcopy.wait()
            x2d = x_vmem[...]
            c_loc = jnp.dot(
                x2d, wdkv_ref[...], preferred_element_type=jnp.float32
            )
            c_buf[...] = c_loc
            c_rdma().start()
            kr_buf[...] = jnp.dot(
                x2d, wkr_ref[...], preferred_element_type=jnp.float32
            )

            c_cat[:, :DC_HALF] = c_loc
            wuk_cat[:, :DC_HALF, :] = wuk_ref[...]
            wuv_cat[:, :DC_HALF, :] = wuv_ref[...]

            c_rdma().wait_recv()

        @pl.when(ck == 0)
        def _build_kv():
            wuk_rdma(h).wait_recv()
            wuv_rdma(h).wait_recv()
            k_buf[h] = jnp.dot(
                c_cat[...], wuk_cat[h], preferred_element_type=jnp.float32
            )

        Kh = k_buf[h]
        Vh = jnp.dot(
            c_cat[...], wuv_cat[h], preferred_element_type=jnp.float32
        )
        x_mine = x_vmem[pl.ds(rid * SR + ck * CH, CH), :]
        Qh = jnp.dot(x_mine, wq_ref[...], preferred_element_type=jnp.float32)
        Qrh = jnp.dot(x_mine, wqr_ref[0], preferred_element_type=jnp.float32)

        scale = (Dh + Dr) ** -0.5
        dn = (((1,), (1,)), ((), ()))
        s = (
            lax.dot_general(Qh, Kh, dn, preferred_element_type=jnp.float32)
            + lax.dot_general(
                Qrh, kr_buf[...], dn, preferred_element_type=jnp.float32
            )
        ) * scale
        m = jnp.max(s, axis=-1, keepdims=True)
        p = jnp.exp(s - m)
        p = p / jnp.sum(p, axis=-1, keepdims=True)
        Oh = jnp.dot(p, Vh, preferred_element_type=jnp.float32)
        contrib = jnp.dot(Oh, wo_ref[...], preferred_element_type=jnp.float32)

        row = pl.ds(ck * CH, CH)

        @pl.when(h == 0)
        def _init():
            acc[row, :] = contrib

        @pl.when(h > 0)
        def _acc():
            acc[row, :] += contrib

        @pl.when(jnp.logical_and(ck == 0, h == H - 1))
        def _push_chunk0():
            local_copy(0).start()
            for i, p in enumerate(peers):
                ag_rdma(0, i, p).start()

        @pl.when(jnp.logical_and(ck == 1, h == H - 1))
        def _push_chunk1_and_drain():
            local_copy(1).start()
            for i, p in enumerate(peers):
                ag_rdma(1, i, p).start()
            for cck in range(NCK):
                local_copy(cck).wait()
                for i, p in enumerate(peers):
                    r = ag_rdma(cck, i, p)
                    r.wait_send()
                    r.wait_recv()
            c_rdma().wait_send()
            for hh in range(H):
                wuk_rdma(hh).wait_send()
                wuv_rdma(hh).wait_send()

    out4 = pl.pallas_call(
        body,
        grid=(NCK, H),
        out_shape=jax.ShapeDtypeStruct((NROW, SR, D), jnp.float32),
        in_specs=[
            pl.BlockSpec(memory_space=pl.ANY),
            pl.BlockSpec((D, DC_HALF), lambda ck, h: (0, 0)),
            pl.BlockSpec((H, DC_HALF, Dh), lambda ck, h: (0, 0, 0)),
            pl.BlockSpec((H, DC_HALF, Dh), lambda ck, h: (0, 0, 0)),
            pl.BlockSpec((D, Dh), lambda ck, h: (0, h)),
            pl.BlockSpec((1, D, Dr), lambda ck, h: (h, 0, 0)),
            pl.BlockSpec((D, Dr), lambda ck, h: (0, 0)),
            pl.BlockSpec((Dh, D), lambda ck, h: (h, 0)),
        ],
        out_specs=pl.BlockSpec(memory_space=pl.ANY),
        scratch_shapes=[
            pltpu.VMEM((S, D), jnp.float32),
            pltpu.VMEM((SR, D), jnp.float32),
            pltpu.VMEM((S, DC_HALF), jnp.float32),
            pltpu.VMEM((S, DC), jnp.float32),
            pltpu.VMEM((H, DC, Dh), jnp.float32),
            pltpu.VMEM((H, DC, Dh), jnp.float32),
            pltpu.VMEM((H, S, Dh), jnp.float32),
            pltpu.VMEM((S, Dr), jnp.float32),
            pltpu.SemaphoreType.DMA,
            pltpu.SemaphoreType.DMA((NCK,)),
            pltpu.SemaphoreType.DMA,
            pltpu.SemaphoreType.DMA,
            pltpu.SemaphoreType.DMA((H,)),
            pltpu.SemaphoreType.DMA((H,)),
            pltpu.SemaphoreType.DMA((H,)),
            pltpu.SemaphoreType.DMA((H,)),
            pltpu.SemaphoreType.DMA((NCK, 3)),
            pltpu.SemaphoreType.DMA((NCK, 3)),
        ],
        compiler_params=pltpu.CompilerParams(
            collective_id=0, vmem_limit_bytes=48 * 1024 * 1024
        ),
    )(x, Wdkv, WukT, WuvT, Wq, WqrT, Wkr, Wo)
    return out4.reshape(B, S, D)


# device time: 115907 ns/iter; 1.2824x vs baseline; 1.2824x over previous
import jax
import jax.numpy as jnp
from jax import lax
from jax.experimental import pallas as pl
from jax.experimental.pallas import tpu as pltpu

B, S, D = 1, 1024, 2048
H, Dh, Dr = 16, 128, 32
DC_HALF = 128
DC = 256
NROW = 4
SR = S // NROW
NCK = 2
CH = SR // NCK
HPS = 2
NG = H // HPS


def kernel(x, Wdkv, Wuk, Wuv, Wq, Wqr, Wkr, Wo):
    WukT = Wuk.reshape(DC_HALF, H, Dh).transpose(1, 2, 0)
    WuvT = Wuv.reshape(DC_HALF, H, Dh).transpose(1, 0, 2)
    WqrT = Wqr.reshape(D, H, Dr).transpose(1, 0, 2)

    def body(
        x_ref, wdkv_ref, wuk_ref, wuv_ref, wq_ref, wqr_ref, wkr_ref, wo_ref,
        out_ref,
        x_vmem, acc, c_buf, c_cat, cT_cat, wuk_cat, wuv_cat, k_buf, kr_buf,
        x_sem, local_sems, c_send_sem, c_recv_sem,
        wk_send_sems, wk_recv_sems, wv_send_sems, wv_recv_sems,
        ag_send_sems, ag_recv_sems,
    ):
        ck = pl.program_id(0)
        g = pl.program_id(1)
        my_x = lax.axis_index("x")
        my_y = lax.axis_index("y")
        rid = 2 * my_x + my_y
        x_nbr = (1 - my_x, my_y)
        y_nbr = (my_x, 1 - my_y)
        diag = (1 - my_x, 1 - my_y)
        peers = (x_nbr, y_nbr, diag)

        def wuk_rdma(hh):
            return pltpu.make_async_remote_copy(
                src_ref=wuk_ref.at[hh],
                dst_ref=wuk_cat.at[hh, :, DC_HALF:],
                send_sem=wk_send_sems.at[hh], recv_sem=wk_recv_sems.at[hh],
                device_id=x_nbr, device_id_type=pl.DeviceIdType.MESH,
            )

        def wuv_rdma(hh):
            return pltpu.make_async_remote_copy(
                src_ref=wuv_ref.at[hh],
                dst_ref=wuv_cat.at[hh, DC_HALF:, :],
                send_sem=wv_send_sems.at[hh], recv_sem=wv_recv_sems.at[hh],
                device_id=x_nbr, device_id_type=pl.DeviceIdType.MESH,
            )

        def c_rdma():
            return pltpu.make_async_remote_copy(
                src_ref=c_buf, dst_ref=c_cat.at[:, DC_HALF:],
                send_sem=c_send_sem, recv_sem=c_recv_sem,
                device_id=x_nbr, device_id_type=pl.DeviceIdType.MESH,
            )

        def ag_rdma(cck, i, p):
            return pltpu.make_async_remote_copy(
                src_ref=acc.at[pl.ds(cck * CH, CH), :],
                dst_ref=out_ref.at[rid, pl.ds(cck * CH, CH), :],
                send_sem=ag_send_sems.at[cck, i],
                recv_sem=ag_recv_sems.at[cck, i],
                device_id=p, device_id_type=pl.DeviceIdType.MESH,
            )

        def local_copy(cck):
            return pltpu.make_async_copy(
                acc.at[pl.ds(cck * CH, CH), :],
                out_ref.at[rid, pl.ds(cck * CH, CH), :],
                local_sems.at[cck],
            )

        @pl.when(jnp.logical_and(ck == 0, g == 0))
        def _exchange():
            barrier_sem = pltpu.get_barrier_semaphore()
            for p in peers:
                pl.semaphore_signal(
                    barrier_sem, inc=1, device_id=p,
                    device_id_type=pl.DeviceIdType.MESH,
                )
            pl.semaphore_wait(barrier_sem, 3)

            for hh in range(HPS):
                wuk_rdma(hh).start()
                wuv_rdma(hh).start()

            xcopy = pltpu.make_async_copy(x_ref.at[0], x_vmem, x_sem)
            xcopy.start()
            xcopy.wait()
            x2d = x_vmem[...]
            c_loc = jnp.dot(
                x2d, wdkv_ref[...], preferred_element_type=jnp.float32
            )
            c_buf[...] = c_loc
            c_rdma().start()
            for hh in range(HPS, H):
                wuk_rdma(hh).start()
                wuv_rdma(hh).start()
            kr_buf[...] = jnp.dot(
                x2d, wkr_ref[...], preferred_element_type=jnp.float32
            ).T

            c_cat[:, :DC_HALF] = c_loc
            wuk_cat[:, :, :DC_HALF] = wuk_ref[...]
            wuv_cat[:, :DC_HALF, :] = wuv_ref[...]

            c_rdma().wait_recv()
            cT_cat[...] = c_cat[...].T

        @pl.when(ck == 0)
        def _build_kv():
            for j in range(HPS):
                hh = g * HPS + j
                wuk_rdma(hh).wait_recv()
                wuv_rdma(hh).wait_recv()
                k_buf[hh] = jnp.dot(
                    wuk_cat[hh], cT_cat[...],
                    preferred_element_type=jnp.float32,
                )

        x_mine = x_vmem[pl.ds(rid * SR + ck * CH, CH), :]
        scale = (Dh + Dr) ** -0.5
        contrib = None
        for j in range(HPS):
            hh = g * HPS + j
            Kh = k_buf[hh]
            Vh = jnp.dot(
                c_cat[...], wuv_cat[hh], preferred_element_type=jnp.float32
            )
            Qh = jnp.dot(
                x_mine, wq_ref[:, j * Dh:(j + 1) * Dh],
                preferred_element_type=jnp.float32,
            )
            Qrh = jnp.dot(
                x_mine, wqr_ref[j], preferred_element_type=jnp.float32
            )
            s = (
                jnp.dot(Qh, Kh, preferred_element_type=jnp.float32)
                + jnp.dot(Qrh, kr_buf[...], preferred_element_type=jnp.float32)
            ) * scale
            p = jnp.exp(s)
            z = jnp.sum(p, axis=-1, keepdims=True)
            Oh = jnp.dot(p, Vh, preferred_element_type=jnp.float32) / z
            part = jnp.dot(
                Oh, wo_ref[j * Dh:(j + 1) * Dh, :],
                preferred_element_type=jnp.float32,
            )
            contrib = part if contrib is None else contrib + part

        row = pl.ds(ck * CH, CH)

        @pl.when(g == 0)
        def _init():
            acc[row, :] = contrib

        @pl.when(g > 0)
        def _acc():
            acc[row, :] += contrib

        @pl.when(jnp.logical_and(ck == 0, g == NG - 1))
        def _push_chunk0():
            local_copy(0).start()
            for i, p in enumerate(peers):
                ag_rdma(0, i, p).start()

        @pl.when(jnp.logical_and(ck == 1, g == NG - 1))
        def _push_chunk1_and_drain():
            local_copy(1).start()
            for i, p in enumerate(peers):
                ag_rdma(1, i, p).start()
            for cck in range(NCK):
                local_copy(cck).wait()
                for i, p in enumerate(peers):
                    r = ag_rdma(cck, i, p)
                    r.wait_send()
                    r.wait_recv()
            c_rdma().wait_send()
            for hh in range(H):
                wuk_rdma(hh).wait_send()
                wuv_rdma(hh).wait_send()

    out4 = pl.pallas_call(
        body,
        grid=(NCK, NG),
        out_shape=jax.ShapeDtypeStruct((NROW, SR, D), jnp.float32),
        in_specs=[
            pl.BlockSpec(memory_space=pl.ANY),
            pl.BlockSpec((D, DC_HALF), lambda ck, h: (0, 0)),
            pl.BlockSpec((H, DC_HALF, Dh), lambda ck, h: (0, 0, 0)),
            pl.BlockSpec((H, DC_HALF, Dh), lambda ck, h: (0, 0, 0)),
            pl.BlockSpec((D, HPS * Dh), lambda ck, g: (0, g)),
            pl.BlockSpec((HPS, D, Dr), lambda ck, g: (g, 0, 0)),
            pl.BlockSpec((D, Dr), lambda ck, g: (0, 0)),
            pl.BlockSpec((HPS * Dh, D), lambda ck, g: (g, 0)),
        ],
        out_specs=pl.BlockSpec(memory_space=pl.ANY),
        scratch_shapes=[
            pltpu.VMEM((S, D), jnp.float32),
            pltpu.VMEM((SR, D), jnp.float32),
            pltpu.VMEM((S, DC_HALF), jnp.float32),
            pltpu.VMEM((S, DC), jnp.float32),
            pltpu.VMEM((DC, S), jnp.float32),
            pltpu.VMEM((H, Dh, DC), jnp.float32),
            pltpu.VMEM((H, DC, Dh), jnp.float32),
            pltpu.VMEM((H, Dh, S), jnp.float32),
            pltpu.VMEM((Dr, S), jnp.float32),
            pltpu.SemaphoreType.DMA,
            pltpu.SemaphoreType.DMA((NCK,)),
            pltpu.SemaphoreType.DMA,
            pltpu.SemaphoreType.DMA,
            pltpu.SemaphoreType.DMA((H,)),
            pltpu.SemaphoreType.DMA((H,)),
            pltpu.SemaphoreType.DMA((H,)),
            pltpu.SemaphoreType.DMA((H,)),
            pltpu.SemaphoreType.DMA((NCK, 3)),
            pltpu.SemaphoreType.DMA((NCK, 3)),
        ],
        compiler_params=pltpu.CompilerParams(
            collective_id=0, vmem_limit_bytes=48 * 1024 * 1024
        ),
    )(x, Wdkv, WukT, WuvT, Wq, WqrT, Wkr, Wo)
    return out4.reshape(B, S, D)
